# baseline (speedup 1.0000x reference)
import functools

import jax
import jax.numpy as jnp
from jax import lax
from jax.experimental import pallas as pl
from jax.experimental.pallas import tpu as pltpu
from jax.experimental.pallas import tpu_sc as plsc

NPTS = 100000
NVOX = 10000
NW = 32
CHUNK = 3136
NPAD = NW * CHUNK
PB = 32
SWIN = 256
GB = 112
GCH = CHUNK // GB
NEG = -3.0e38


def _dot(a, b):
    return lax.dot_general(a, b, (((1,), (0,)), ((), ())),
                           preferred_element_type=jnp.float32)


def _stage1_body(feat_ref, w1x, b1x, w2x, b2x, w1r, b1r, w2r, b2r,
                 wax, bax, war, bar, o_ref):
    px = jnp.maximum(_dot(feat_ref[...], w1x[...]) + b1x[...], 0.0)
    px = jnp.maximum(_dot(px, w2x[...]) + b2x[...], 0.0)
    pr = jnp.maximum(_dot(feat_ref[...], w1r[...]) + b1r[...], 0.0)
    pr = jnp.maximum(_dot(pr, w2r[...]) + b2r[...], 0.0)
    comb = jnp.concatenate([px, pr], axis=1)
    ax = jax.nn.sigmoid(_dot(comb, wax[...]) + bax[...])
    ar = jax.nn.sigmoid(_dot(comb, war[...]) + bar[...])
    rowix = (lax.broadcasted_iota(jnp.int32, (px.shape[0], 128), 0)
             + pl.program_id(0) * px.shape[0])
    valid = rowix < NPTS
    o_ref[...] = jnp.where(
        valid,
        jnp.concatenate([px, px * ax, pr, pr * ar], axis=1),
        NEG).astype(jnp.bfloat16)


def _stage1(feat, w1x, b1x, w2x, b2x, w1r, b1r, w2r, b2r, wax, bax, war, bar):
    blk = 2048
    nb = NPAD // blk
    full = lambda s: pl.BlockSpec(s, lambda i: (0, 0))
    return pl.pallas_call(
        _stage1_body,
        grid=(nb,),
        in_specs=[
            pl.BlockSpec((blk, 6), lambda i: (i, 0)),
            full((6, 64)), full((1, 64)), full((64, 32)), full((1, 32)),
            full((6, 64)), full((1, 64)), full((64, 32)), full((1, 32)),
            full((64, 32)), full((1, 32)), full((64, 32)), full((1, 32)),
        ],
        out_specs=pl.BlockSpec((blk, 128), lambda i: (i, 0)),
        out_shape=jax.ShapeDtypeStruct((NPAD, 128), jnp.bfloat16),
    )(feat, w1x, b1x, w2x, b2x, w1r, b1r, w2r, b2r, wax, bax, war, bar)


def _corrected(x_ref, pre_ref, ids_sub_ref, ids_lane_ref, blk, i):
    pre_m = jnp.maximum(pre_ref[...].astype(jnp.float32), NEG)
    ids_s = ids_sub_ref[...]
    ids_l = ids_lane_ref[...]
    comb = pre_m
    k = 1
    while k < NW:
        comb_sh = jnp.concatenate(
            [comb[k:], jnp.full((k, comb.shape[1]), NEG, jnp.float32)], axis=0)
        cs_sh = jnp.concatenate(
            [ids_s[k:], jnp.full((k, 1), -1, jnp.int32)], axis=0)
        eq = ids_s == cs_sh
        comb = jnp.maximum(comb, jnp.where(eq, comb_sh, NEG))
        k *= 2
    prev_s = jnp.concatenate(
        [jnp.full((1, 1), -1, jnp.int32), ids_s[:-1]], axis=0)
    use_s = jnp.logical_and(ids_s >= 0, ids_s != prev_s)
    prev_l = jnp.concatenate(
        [jnp.full((1, 1), -1, jnp.int32), ids_l[:, :-1]], axis=1)
    use_l = jnp.logical_and(ids_l >= 0, ids_l != prev_l)
    eff = jnp.where(use_s, comb, NEG)
    eff_ids = jnp.where(use_l, ids_l, -1)
    rows = lax.broadcasted_iota(jnp.int32, (blk, NW), 0) + i * blk
    onehot = (rows == eff_ids).astype(jnp.float32)
    overlay = _dot(onehot, eff)
    maskc = jnp.sum(onehot, axis=1, keepdims=True)
    x = x_ref[...].astype(jnp.float32)
    return jnp.where(maskc > 0.0, jnp.maximum(x, overlay), x)


def _dense_corr_body(x_ref, pre_ref, ids_sub_ref, ids_lane_ref, w_ref, b_ref,
                     o_ref, *, blk):
    x = _corrected(x_ref, pre_ref, ids_sub_ref, ids_lane_ref, blk,
                   pl.program_id(0))
    o_ref[...] = jnp.maximum(_dot(x, w_ref[...]) + b_ref[...], 0.0)


def _dense_corr(x, pre, ids_sub, ids_lane, w, b, blk):
    n, kk = x.shape
    ko, d = w.shape
    nb = n // blk
    full = lambda s: pl.BlockSpec(s, lambda i: (0, 0))
    return pl.pallas_call(
        functools.partial(_dense_corr_body, blk=blk),
        grid=(nb,),
        in_specs=[
            pl.BlockSpec((blk, kk), lambda i: (i, 0)),
            full((NW, kk)), full((NW, 1)), full((1, NW)),
            full((ko, d)), full((1, d)),
        ],
        out_specs=pl.BlockSpec((blk, d), lambda i: (i, 0)),
        out_shape=jax.ShapeDtypeStruct((n, d), jnp.float32),
    )(x, pre, ids_sub, ids_lane, w, b)


def _dense_corr2_body(xa_ref, xb_ref, prea_ref, preb_ref, ids_sub_ref,
                      ids_lane_ref, wa_ref, wb_ref, b_ref, o_ref, *, blk):
    i = pl.program_id(0)
    xa = _corrected(xa_ref, prea_ref, ids_sub_ref, ids_lane_ref, blk, i)
    xb = _corrected(xb_ref, preb_ref, ids_sub_ref, ids_lane_ref, blk, i)
    o_ref[...] = jnp.maximum(
        _dot(xa, wa_ref[...]) + _dot(xb, wb_ref[...]) + b_ref[...], 0.0)


def _dense_corr2(xa, xb, prea, preb, ids_sub, ids_lane, wa, wb, b, blk):
    n, kk = xa.shape
    ko, d = wa.shape
    nb = n // blk
    full = lambda s: pl.BlockSpec(s, lambda i: (0, 0))
    return pl.pallas_call(
        functools.partial(_dense_corr2_body, blk=blk),
        grid=(nb,),
        in_specs=[
            pl.BlockSpec((blk, kk), lambda i: (i, 0)),
            pl.BlockSpec((blk, kk), lambda i: (i, 0)),
            full((NW, kk)), full((NW, kk)), full((NW, 1)), full((1, NW)),
            full((ko, d)), full((ko, d)), full((1, d)),
        ],
        out_specs=pl.BlockSpec((blk, d), lambda i: (i, 0)),
        out_shape=jax.ShapeDtypeStruct((n, d), jnp.float32),
    )(xa, xb, prea, preb, ids_sub, ids_lane, wa, wb, b)


def _stage3_body(pg_ref, pf2_ref, w3a_ref, w3b_ref, b3_ref, w4_ref, b4_ref, o_ref):
    bf = jnp.bfloat16
    blk = pg_ref.shape[0]
    pf4 = jnp.maximum(
        _dot(pg_ref[...].astype(bf), w3a_ref[...])
        + _dot(jnp.maximum(pf2_ref[...], NEG * 0.01).astype(bf), w3b_ref[...])
        + b3_ref[...], 0.0)
    rowix = (lax.broadcasted_iota(jnp.int32, (blk, 256), 0)
             + pl.program_id(0) * blk)
    valid = rowix < NPTS
    pf5 = jnp.where(
        valid, jnp.maximum(_dot(pf4.astype(bf), w4_ref[...]) + b4_ref[...],
                           0.0), NEG).astype(bf)
    o_ref[0, :, :] = pf5[:, :128]
    o_ref[1, :, :] = pf5[:, 128:]


def _stage3(pg, pf2, w3a, w3b, b3, w4, b4):
    blk = 1024
    nb = NPAD // blk
    full = lambda s: pl.BlockSpec(s, lambda i: (0, 0))
    return pl.pallas_call(
        _stage3_body,
        grid=(nb,),
        in_specs=[
            pl.BlockSpec((blk, 128), lambda i: (i, 0)),
            pl.BlockSpec((blk, 128), lambda i: (i, 0)),
            full((128, 256)), full((128, 256)), full((1, 256)),
            full((256, 256)), full((1, 256)),
        ],
        out_specs=pl.BlockSpec((2, blk, 128), lambda i: (0, i, 0)),
        out_shape=jax.ShapeDtypeStruct((2, NPAD, 128), jnp.bfloat16),
    )(pg, pf2, w3a, w3b, b3, w4, b4)


def _make_segmax(D):
    NH = D // 128
    nvec = D // 32
    nbat2 = CHUNK // (2 * PB)
    mesh = plsc.VectorSubcoreMesh(core_axis_name="c", subcore_axis_name="s")

    @functools.partial(
        pl.kernel,
        out_type=(jax.ShapeDtypeStruct((NVOX * D,), jnp.bfloat16),
                  jax.ShapeDtypeStruct((NW * D,), jnp.bfloat16),
                  jax.ShapeDtypeStruct((NW * 16,), jnp.int32)),
        mesh=mesh,
        compiler_params=pltpu.CompilerParams(use_tc_tiling_on_sc=False),
        scratch_types=[
            pltpu.VMEM((CHUNK + 32,), jnp.int32),
            pltpu.VMEM((16,), jnp.int32),
            pltpu.VMEM((2 * PB * D,), jnp.bfloat16),
            pltpu.VMEM((SWIN * D,), jnp.bfloat16),
            pltpu.SemaphoreType.DMA,
            pltpu.SemaphoreType.DMA,
            pltpu.SemaphoreType.DMA,
        ],
    )
    def segmax(idx_hbm, feat_hbm, out_hbm, pre_hbm, preid_hbm,
               idx_v, pid_v, rows, stag, sem0, sem1, semf):
        wid = lax.axis_index("s") * 2 + lax.axis_index("c")
        start = pl.multiple_of(wid * CHUNK, 64)

        pltpu.sync_copy(idx_hbm.at[pl.ds(start, CHUNK + 16)],
                        idx_v.at[pl.ds(0, CHUNK + 16)])
        previd = idx_v[pl.ds(0, 16)][15]
        lo = previd + 1
        hivec = idx_v[pl.ds(CHUNK, 16)]
        hi = jnp.minimum(hivec[15] + 1, NVOX)

        zvec = jnp.zeros((32,), jnp.bfloat16)

        def _zero_stag():
            def zb(r, _):
                ro = pl.multiple_of(r * D, 32)
                for k in range(nvec):
                    stag[pl.ds(ro + k * 32, 32)] = zvec
                return 0
            lax.fori_loop(0, SWIN, zb, 0)

        _zero_stag()

        def _flush_full(wl):
            wo = pl.multiple_of(wl * 128, 8)
            for h in range(NH):
                for g in range(SWIN // 8):
                    pltpu.async_copy(
                        stag.at[pl.ds(h * SWIN * 128 + g * 1024, 1024)],
                        out_hbm.at[pl.ds(h * NVOX * 128 + wo + g * 1024,
                                         1024)], semf)
            for g in range(NH * (SWIN // 8)):
                pltpu.make_async_copy(stag.at[pl.ds(0, 1024)],
                                      out_hbm.at[pl.ds(0, 1024)], semf).wait()
            _zero_stag()

        sems = (sem0, sem1)
        for h in range(NH):
            pltpu.async_copy(
                feat_hbm.at[pl.ds(h * NPAD * 128 + start * 128, PB * 128)],
                rows.at[pl.ds(h * 2 * PB * 128, PB * 128)], sem0)
            pltpu.async_copy(
                feat_hbm.at[pl.ds(h * NPAD * 128 + (start + PB) * 128,
                                  PB * 128)],
                rows.at[pl.ds(h * 2 * PB * 128 + PB * 128, PB * 128)], sem1)

        def _advance(closing, cur_id, win_lo):
            njump = jnp.where(closing, (cur_id - win_lo) // SWIN, 0)

            def wbody(i, wl):
                _flush_full(wl)
                return wl + SWIN

            return lax.fori_loop(0, njump, wbody, win_lo)

        def _close(closing, cur_id, win_lo, running):
            win_lo = _advance(closing, cur_id, win_lo)

            @pl.when(closing)
            def _():
                ro = pl.multiple_of((cur_id - win_lo) * 128, 32)
                for k in range(nvec):
                    stag[pl.ds((k // 4) * SWIN * 128 + ro + (k % 4) * 32,
                               32)] = running[k]

            return win_lo

        def _proc_point(idv, row, cur_id, win_lo, running, pre):
            is_pre = idv == previd
            is_proc = jnp.logical_and(idv >= lo, idv < hi)
            is_new = jnp.logical_and(is_proc, idv != cur_id)
            win_lo = _close(jnp.logical_and(is_new, cur_id >= lo), cur_id,
                            win_lo, running)
            new_running = []
            new_pre = []
            for k in range(nvec):
                new_running.append(
                    jnp.where(is_new, row[k], jnp.maximum(running[k], row[k])))
                new_pre.append(jnp.where(is_pre, jnp.maximum(pre[k], row[k]),
                                         pre[k]))
            cur_id = jnp.where(is_new, idv, cur_id)
            return cur_id, win_lo, new_running, new_pre

        def _proc_batch(bi, buf, carry):
            pltpu.make_async_copy(feat_hbm.at[pl.ds(0, PB * D)],
                                  rows.at[pl.ds(0, PB * D)], sems[buf]).wait()

            def pbody(j, c):
                cur_id, win_lo, running, pre = c
                idv = idx_v[pl.ds(16 + bi * PB + j, 16)][0]
                ro = pl.multiple_of((buf * PB + j) * 128, 32)
                row = [rows[pl.ds((k // 4) * 2 * PB * 128 + ro
                                  + (k % 4) * 32, 32)] for k in range(nvec)]
                return _proc_point(idv, row, cur_id, win_lo, running, pre)

            carry = lax.fori_loop(0, PB, pbody, carry)
            @pl.when(bi + 2 < CHUNK // PB)
            def _():
                for h in range(NH):
                    pltpu.async_copy(
                        feat_hbm.at[pl.ds(
                            h * NPAD * 128 + (start + (bi + 2) * PB) * 128,
                            PB * 128)],
                        rows.at[pl.ds(h * 2 * PB * 128 + buf * PB * 128,
                                      PB * 128)], sems[buf])
            return carry

        def _main(b2, carry):
            carry = _proc_batch(2 * b2, 0, carry)
            carry = _proc_batch(2 * b2 + 1, 1, carry)
            return carry

        neg0 = [jnp.full((32,), NEG, jnp.bfloat16) for _ in range(nvec)]
        cur_id, win_lo, running, pre = lax.fori_loop(
            0, nbat2, _main,
            (jnp.int32(-2), lo, list(neg0), list(neg0)))

        win_lo = _close(cur_id >= lo, cur_id, win_lo, running)

        cnt = jnp.maximum(hi - win_lo, 0)
        n8 = cnt // 8
        wo = pl.multiple_of(win_lo * 128, 8)

        def f8(g, _):
            for h in range(NH):
                pltpu.async_copy(
                    stag.at[pl.ds(h * SWIN * 128 + g * 1024, 1024)],
                    out_hbm.at[pl.ds(h * NVOX * 128 + wo + g * 1024, 1024)],
                    semf)
            return 0

        lax.fori_loop(0, n8, f8, 0)

        def f1(r, _):
            so = pl.multiple_of((n8 * 8 + r) * 128, 8)
            for h in range(NH):
                pltpu.async_copy(
                    stag.at[pl.ds(h * SWIN * 128 + so, 128)],
                    out_hbm.at[pl.ds(h * NVOX * 128 + wo + so, 128)], semf)
            return 0

        lax.fori_loop(0, cnt - n8 * 8, f1, 0)

        for k in range(nvec):
            rows[pl.ds(k * 32, 32)] = pre[k]
        pid_v[pl.ds(0, 16)] = jnp.full((16,), previd, jnp.int32)
        pltpu.async_copy(rows.at[pl.ds(0, D)],
                         pre_hbm.at[pl.ds(wid * D, D)], semf)
        pltpu.async_copy(pid_v.at[pl.ds(0, 16)],
                         preid_hbm.at[pl.ds(wid * 16, 16)], semf)

        def d8(g, _):
            for h in range(NH):
                pltpu.make_async_copy(stag.at[pl.ds(0, 1024)],
                                      out_hbm.at[pl.ds(0, 1024)], semf).wait()
            return 0

        lax.fori_loop(0, n8, d8, 0)

        def d1(r, _):
            for h in range(NH):
                pltpu.make_async_copy(stag.at[pl.ds(0, 128)],
                                      out_hbm.at[pl.ds(0, 128)], semf).wait()
            return 0

        lax.fori_loop(0, cnt - n8 * 8, d1, 0)
        pltpu.make_async_copy(rows.at[pl.ds(0, D)],
                              pre_hbm.at[pl.ds(0, D)], semf).wait()
        pltpu.make_async_copy(pid_v.at[pl.ds(0, 16)],
                              preid_hbm.at[pl.ds(0, 16)], semf).wait()

    return segmax


_segmax128 = _make_segmax(128)
_segmax256 = _make_segmax(256)


def _gather_kernel(idx2_hbm, table_hbm, out_hbm, idx_v, rows, g0, g1, p0, p1):
    wid = lax.axis_index("s") * 2 + lax.axis_index("c")
    base = pl.multiple_of(wid * CHUNK, 64)
    pltpu.sync_copy(idx2_hbm.at[wid], idx_v)
    gsems = (g0, g1)
    psems = (p0, p1)

    def _start(g, buf):
        pltpu.async_copy(table_hbm.at[idx_v.at[g]], rows.at[buf], gsems[buf])

    def _finish(g, buf):
        pltpu.make_async_copy(table_hbm.at[idx_v.at[0]], rows.at[buf],
                              gsems[buf]).wait()
        pltpu.async_copy(rows.at[buf], out_hbm.at[pl.ds(base + g * GB, GB)],
                         psems[buf])

    _start(0, 0)
    _start(1, 1)

    def body(h, _):
        for par in range(2):
            g = 2 * h + par
            _finish(g, par)
            pltpu.make_async_copy(rows.at[par], out_hbm.at[pl.ds(0, GB)],
                                  psems[par]).wait()

            @pl.when(g + 2 < GCH)
            def _():
                _start(g + 2, par)
        return 0

    lax.fori_loop(0, GCH // 2, body, 0)


def _gather(idx2, table):
    mesh = plsc.VectorSubcoreMesh(core_axis_name="c", subcore_axis_name="s")
    k = functools.partial(
        pl.kernel,
        out_type=jax.ShapeDtypeStruct((NPAD, 128), jnp.float32),
        mesh=mesh,
        compiler_params=pltpu.CompilerParams(use_tc_tiling_on_sc=False),
        scratch_types=[
            pltpu.VMEM((GCH, GB), jnp.int32),
            pltpu.VMEM((2, GB, 128), jnp.float32),
            pltpu.SemaphoreType.DMA,
            pltpu.SemaphoreType.DMA,
            pltpu.SemaphoreType.DMA,
            pltpu.SemaphoreType.DMA,
        ],
    )(_gather_kernel)
    return k(idx2, table)


def kernel(inp_feat, vox2point_idx, W1x, b1x, W2x, b2x, W1r, b1r, W2r, b2r,
           Wax, bax, War, bar, Wv1, bv1, W3, b3, W4, b4, Wv2, bv2):
    f32 = jnp.float32
    bf16 = jnp.bfloat16
    idx = vox2point_idx.astype(jnp.int32)
    idx_pad = jnp.concatenate([
        jnp.full((16,), -1, jnp.int32), idx,
        jnp.full((NPAD - NPTS,), NVOX, jnp.int32)])
    idx2 = jnp.minimum(idx_pad[16:], NVOX - 1).reshape(NW, GCH, GB)

    feat = jnp.zeros((NPAD, 6), f32).at[:NPTS].set(inp_feat)
    z34 = jnp.zeros((3, 64), f32)
    w1x6 = jnp.concatenate([W1x.astype(f32), z34], axis=0)
    w1r6 = jnp.concatenate([z34, W1r.astype(f32)], axis=0)

    r2 = lambda b: b.reshape(1, -1).astype(f32)

    pf2 = _stage1(feat, w1x6, r2(b1x), W2x, r2(b2x), w1r6, r2(b1r),
                  W2r, r2(b2r), Wax, r2(bax), War, r2(bar))

    segraw, pre1, preid1 = _segmax128(idx_pad, pf2.reshape(-1))
    segraw = segraw.reshape(NVOX, 128)
    ids1 = preid1.reshape(NW, 16)
    seg = _dense_corr(segraw, pre1.reshape(NW, 128), ids1[:, :1],
                      ids1[:, 0].reshape(1, NW), Wv1, r2(bv1), 1000)

    pg = _gather(idx2, seg)

    pf5 = _stage3(pg, pf2, W3[:128].astype(bf16), W3[128:].astype(bf16),
                  r2(b3), W4.astype(bf16), r2(b4))

    seg2f, pre2, preid2 = _segmax256(idx_pad, pf5.reshape(-1))
    seg2 = seg2f.reshape(2, NVOX, 128)
    pre2m = pre2.reshape(NW, 256)
    ids2 = preid2.reshape(NW, 16)
    out = _dense_corr2(seg2[0], seg2[1], pre2m[:, :128], pre2m[:, 128:],
                       ids2[:, :1], ids2[:, 0].reshape(1, NW),
                       Wv2[:128], Wv2[128:], r2(bv2), 1000)
    return out


# speedup vs baseline: 1.3755x; 1.3755x over previous
import functools

import jax
import jax.numpy as jnp
from jax import lax
from jax.experimental import pallas as pl
from jax.experimental.pallas import tpu as pltpu
from jax.experimental.pallas import tpu_sc as plsc

NPTS = 100000
NVOX = 10000
NW = 32
CHUNK = 3136
NPAD = NW * CHUNK
PB = 32
SWIN = 256
GB = 112
GCH = CHUNK // GB
NEG = -3.0e38


def _dot(a, b):
    return lax.dot_general(a, b, (((1,), (0,)), ((), ())),
                           preferred_element_type=jnp.float32)


def _stage1_body(feat_ref, w1x, b1x, w2x, b2x, w1r, b1r, w2r, b2r,
                 wax, bax, war, bar, o_ref):
    px = jnp.maximum(_dot(feat_ref[...], w1x[...]) + b1x[...], 0.0)
    px = jnp.maximum(_dot(px, w2x[...]) + b2x[...], 0.0)
    pr = jnp.maximum(_dot(feat_ref[...], w1r[...]) + b1r[...], 0.0)
    pr = jnp.maximum(_dot(pr, w2r[...]) + b2r[...], 0.0)
    comb = jnp.concatenate([px, pr], axis=1)
    ax = jax.nn.sigmoid(_dot(comb, wax[...]) + bax[...])
    ar = jax.nn.sigmoid(_dot(comb, war[...]) + bar[...])
    rowix = (lax.broadcasted_iota(jnp.int32, (px.shape[0], 128), 0)
             + pl.program_id(0) * px.shape[0])
    valid = rowix < NPTS
    o_ref[...] = jnp.where(
        valid,
        jnp.concatenate([px, px * ax, pr, pr * ar], axis=1),
        NEG)


def _stage1(feat, w1x, b1x, w2x, b2x, w1r, b1r, w2r, b2r, wax, bax, war, bar):
    blk = 2048
    nb = NPAD // blk
    full = lambda s: pl.BlockSpec(s, lambda i: (0, 0))
    return pl.pallas_call(
        _stage1_body,
        grid=(nb,),
        in_specs=[
            pl.BlockSpec((blk, 6), lambda i: (i, 0)),
            full((6, 64)), full((1, 64)), full((64, 32)), full((1, 32)),
            full((6, 64)), full((1, 64)), full((64, 32)), full((1, 32)),
            full((64, 32)), full((1, 32)), full((64, 32)), full((1, 32)),
        ],
        out_specs=pl.BlockSpec((blk, 128), lambda i: (i, 0)),
        out_shape=jax.ShapeDtypeStruct((NPAD, 128), jnp.float32),
    )(feat, w1x, b1x, w2x, b2x, w1r, b1r, w2r, b2r, wax, bax, war, bar)


def _corrected(x_ref, pre_ref, ids_sub_ref, ids_lane_ref, blk, i):
    pre_m = jnp.maximum(pre_ref[...].astype(jnp.float32), NEG)
    ids_s = ids_sub_ref[...]
    ids_l = ids_lane_ref[...]
    comb = pre_m
    k = 1
    while k < NW:
        comb_sh = jnp.concatenate(
            [comb[k:], jnp.full((k, comb.shape[1]), NEG, jnp.float32)], axis=0)
        cs_sh = jnp.concatenate(
            [ids_s[k:], jnp.full((k, 1), -1, jnp.int32)], axis=0)
        eq = ids_s == cs_sh
        comb = jnp.maximum(comb, jnp.where(eq, comb_sh, NEG))
        k *= 2
    prev_s = jnp.concatenate(
        [jnp.full((1, 1), -1, jnp.int32), ids_s[:-1]], axis=0)
    use_s = jnp.logical_and(ids_s >= 0, ids_s != prev_s)
    prev_l = jnp.concatenate(
        [jnp.full((1, 1), -1, jnp.int32), ids_l[:, :-1]], axis=1)
    use_l = jnp.logical_and(ids_l >= 0, ids_l != prev_l)
    eff = jnp.where(use_s, comb, NEG)
    eff_ids = jnp.where(use_l, ids_l, -1)
    rows = lax.broadcasted_iota(jnp.int32, (blk, NW), 0) + i * blk
    onehot = (rows == eff_ids).astype(jnp.float32)
    overlay = _dot(onehot, eff)
    maskc = jnp.sum(onehot, axis=1, keepdims=True)
    x = x_ref[...].astype(jnp.float32)
    return jnp.where(maskc > 0.0, jnp.maximum(x, overlay), x)


def _dense_corr_body(x_ref, pre_ref, ids_sub_ref, ids_lane_ref, w_ref, b_ref,
                     o_ref, *, blk):
    x = _corrected(x_ref, pre_ref, ids_sub_ref, ids_lane_ref, blk,
                   pl.program_id(0))
    o_ref[...] = jnp.maximum(_dot(x, w_ref[...]) + b_ref[...], 0.0)


def _dense_corr(x, pre, ids_sub, ids_lane, w, b, blk):
    n, kk = x.shape
    ko, d = w.shape
    nb = n // blk
    full = lambda s: pl.BlockSpec(s, lambda i: (0, 0))
    return pl.pallas_call(
        functools.partial(_dense_corr_body, blk=blk),
        grid=(nb,),
        in_specs=[
            pl.BlockSpec((blk, kk), lambda i: (i, 0)),
            full((NW, kk)), full((NW, 1)), full((1, NW)),
            full((ko, d)), full((1, d)),
        ],
        out_specs=pl.BlockSpec((blk, d), lambda i: (i, 0)),
        out_shape=jax.ShapeDtypeStruct((n, d), jnp.float32),
    )(x, pre, ids_sub, ids_lane, w, b)


def _dense_corr2_body(xa_ref, xb_ref, prea_ref, preb_ref, ids_sub_ref,
                      ids_lane_ref, wa_ref, wb_ref, b_ref, o_ref, *, blk):
    i = pl.program_id(0)
    xa = _corrected(xa_ref, prea_ref, ids_sub_ref, ids_lane_ref, blk, i)
    xb = _corrected(xb_ref, preb_ref, ids_sub_ref, ids_lane_ref, blk, i)
    o_ref[...] = jnp.maximum(
        _dot(xa, wa_ref[...]) + _dot(xb, wb_ref[...]) + b_ref[...], 0.0)


def _dense_corr2(xa, xb, prea, preb, ids_sub, ids_lane, wa, wb, b, blk):
    n, kk = xa.shape
    ko, d = wa.shape
    nb = n // blk
    full = lambda s: pl.BlockSpec(s, lambda i: (0, 0))
    return pl.pallas_call(
        functools.partial(_dense_corr2_body, blk=blk),
        grid=(nb,),
        in_specs=[
            pl.BlockSpec((blk, kk), lambda i: (i, 0)),
            pl.BlockSpec((blk, kk), lambda i: (i, 0)),
            full((NW, kk)), full((NW, kk)), full((NW, 1)), full((1, NW)),
            full((ko, d)), full((ko, d)), full((1, d)),
        ],
        out_specs=pl.BlockSpec((blk, d), lambda i: (i, 0)),
        out_shape=jax.ShapeDtypeStruct((n, d), jnp.float32),
    )(xa, xb, prea, preb, ids_sub, ids_lane, wa, wb, b)


def _stage3_body(pg_ref, pf2_ref, w3a_ref, w3b_ref, b3_ref, w4_ref, b4_ref, o_ref):
    bf = jnp.bfloat16
    blk = pg_ref.shape[0]
    pf4 = jnp.maximum(
        _dot(pg_ref[...].astype(bf), w3a_ref[...])
        + _dot(jnp.maximum(pf2_ref[...], NEG * 0.01).astype(bf), w3b_ref[...])
        + b3_ref[...], 0.0)
    rowix = (lax.broadcasted_iota(jnp.int32, (blk, 256), 0)
             + pl.program_id(0) * blk)
    valid = rowix < NPTS
    pf5 = jnp.where(
        valid, jnp.maximum(_dot(pf4.astype(bf), w4_ref[...]) + b4_ref[...],
                           0.0), NEG)
    o_ref[0, :, :] = pf5[:, :128]
    o_ref[1, :, :] = pf5[:, 128:]


def _stage3(pg, pf2, w3a, w3b, b3, w4, b4):
    blk = 1024
    nb = NPAD // blk
    full = lambda s: pl.BlockSpec(s, lambda i: (0, 0))
    return pl.pallas_call(
        _stage3_body,
        grid=(nb,),
        in_specs=[
            pl.BlockSpec((blk, 128), lambda i: (i, 0)),
            pl.BlockSpec((blk, 128), lambda i: (i, 0)),
            full((128, 256)), full((128, 256)), full((1, 256)),
            full((256, 256)), full((1, 256)),
        ],
        out_specs=pl.BlockSpec((2, blk, 128), lambda i: (0, i, 0)),
        out_shape=jax.ShapeDtypeStruct((2, NPAD, 128), jnp.float32),
    )(pg, pf2, w3a, w3b, b3, w4, b4)


def _make_segmax(D):
    NH = D // 128
    nvec = D // 16
    nbat2 = CHUNK // (2 * PB)
    mesh = plsc.VectorSubcoreMesh(core_axis_name="c", subcore_axis_name="s")

    @functools.partial(
        pl.kernel,
        out_type=(jax.ShapeDtypeStruct((NVOX * D,), jnp.float32),
                  jax.ShapeDtypeStruct((NW * D,), jnp.float32),
                  jax.ShapeDtypeStruct((NW * 16,), jnp.int32)),
        mesh=mesh,
        compiler_params=pltpu.CompilerParams(use_tc_tiling_on_sc=False),
        scratch_types=[
            pltpu.VMEM((CHUNK + 32,), jnp.int32),
            pltpu.VMEM((16,), jnp.int32),
            pltpu.VMEM((2 * PB * D,), jnp.float32),
            pltpu.VMEM((SWIN * D,), jnp.float32),
            pltpu.SemaphoreType.DMA,
            pltpu.SemaphoreType.DMA,
            pltpu.SemaphoreType.DMA,
        ],
    )
    def segmax(idx_hbm, feat_hbm, out_hbm, pre_hbm, preid_hbm,
               idx_v, pid_v, rows, stag, sem0, sem1, semf):
        wid = lax.axis_index("s") * 2 + lax.axis_index("c")
        start = pl.multiple_of(wid * CHUNK, 64)

        pltpu.sync_copy(idx_hbm.at[pl.ds(start, CHUNK + 16)],
                        idx_v.at[pl.ds(0, CHUNK + 16)])
        previd = idx_v[pl.ds(0, 16)][15]
        lo = previd + 1
        hivec = idx_v[pl.ds(CHUNK, 16)]
        hi = jnp.minimum(hivec[15] + 1, NVOX)

        zvec = jnp.zeros((16,), jnp.float32)

        def _zero_stag():
            def zb(r, _):
                ro = pl.multiple_of(r * D, 16)
                for k in range(nvec):
                    stag[pl.ds(ro + k * 16, 16)] = zvec
                return 0
            lax.fori_loop(0, SWIN, zb, 0)

        _zero_stag()

        def _flush_full(wl):
            wo = pl.multiple_of(wl * 128, 8)
            for h in range(NH):
                for g in range(SWIN // 8):
                    pltpu.async_copy(
                        stag.at[pl.ds(h * SWIN * 128 + g * 1024, 1024)],
                        out_hbm.at[pl.ds(h * NVOX * 128 + wo + g * 1024,
                                         1024)], semf)
            for g in range(NH * (SWIN // 8)):
                pltpu.make_async_copy(stag.at[pl.ds(0, 1024)],
                                      out_hbm.at[pl.ds(0, 1024)], semf).wait()
            _zero_stag()

        sems = (sem0, sem1)
        for h in range(NH):
            pltpu.async_copy(
                feat_hbm.at[pl.ds(h * NPAD * 128 + start * 128, PB * 128)],
                rows.at[pl.ds(h * 2 * PB * 128, PB * 128)], sem0)
            pltpu.async_copy(
                feat_hbm.at[pl.ds(h * NPAD * 128 + (start + PB) * 128,
                                  PB * 128)],
                rows.at[pl.ds(h * 2 * PB * 128 + PB * 128, PB * 128)], sem1)

        def _advance(closing, cur_id, win_lo):
            njump = jnp.where(closing, (cur_id - win_lo) // SWIN, 0)

            def wbody(i, wl):
                _flush_full(wl)
                return wl + SWIN

            return lax.fori_loop(0, njump, wbody, win_lo)

        def _close(closing, cur_id, win_lo, running):
            win_lo = _advance(closing, cur_id, win_lo)

            @pl.when(closing)
            def _():
                ro = pl.multiple_of((cur_id - win_lo) * 128, 16)
                for k in range(nvec):
                    stag[pl.ds((k // 8) * SWIN * 128 + ro + (k % 8) * 16,
                               16)] = running[k]

            return win_lo

        def _proc_point(idv, row, cur_id, win_lo, running, pre):
            is_pre = idv == previd
            is_proc = jnp.logical_and(idv >= lo, idv < hi)
            is_new = jnp.logical_and(is_proc, idv != cur_id)
            win_lo = _close(jnp.logical_and(is_new, cur_id >= lo), cur_id,
                            win_lo, running)
            new_running = []
            new_pre = []
            for k in range(nvec):
                new_running.append(
                    jnp.where(is_new, row[k], jnp.maximum(running[k], row[k])))
                new_pre.append(jnp.where(is_pre, jnp.maximum(pre[k], row[k]),
                                         pre[k]))
            cur_id = jnp.where(is_new, idv, cur_id)
            return cur_id, win_lo, new_running, new_pre

        def _proc_batch(bi, buf, carry):
            pltpu.make_async_copy(feat_hbm.at[pl.ds(0, PB * D)],
                                  rows.at[pl.ds(0, PB * D)], sems[buf]).wait()

            def pbody(j, c):
                cur_id, win_lo, running, pre = c
                idv = idx_v[pl.ds(16 + bi * PB + j, 16)][0]
                ro = pl.multiple_of((buf * PB + j) * 128, 16)
                row = [rows[pl.ds((k // 8) * 2 * PB * 128 + ro
                                  + (k % 8) * 16, 16)] for k in range(nvec)]
                return _proc_point(idv, row, cur_id, win_lo, running, pre)

            carry = lax.fori_loop(0, PB, pbody, carry)
            @pl.when(bi + 2 < CHUNK // PB)
            def _():
                for h in range(NH):
                    pltpu.async_copy(
                        feat_hbm.at[pl.ds(
                            h * NPAD * 128 + (start + (bi + 2) * PB) * 128,
                            PB * 128)],
                        rows.at[pl.ds(h * 2 * PB * 128 + buf * PB * 128,
                                      PB * 128)], sems[buf])
            return carry

        def _main(b2, carry):
            carry = _proc_batch(2 * b2, 0, carry)
            carry = _proc_batch(2 * b2 + 1, 1, carry)
            return carry

        neg0 = [jnp.full((16,), NEG, jnp.float32) for _ in range(nvec)]
        cur_id, win_lo, running, pre = lax.fori_loop(
            0, nbat2, _main,
            (jnp.int32(-2), lo, list(neg0), list(neg0)))

        win_lo = _close(cur_id >= lo, cur_id, win_lo, running)

        cnt = jnp.maximum(hi - win_lo, 0)
        n8 = cnt // 8
        wo = pl.multiple_of(win_lo * 128, 8)

        def f8(g, _):
            for h in range(NH):
                pltpu.async_copy(
                    stag.at[pl.ds(h * SWIN * 128 + g * 1024, 1024)],
                    out_hbm.at[pl.ds(h * NVOX * 128 + wo + g * 1024, 1024)],
                    semf)
            return 0

        lax.fori_loop(0, n8, f8, 0)

        def f1(r, _):
            so = pl.multiple_of((n8 * 8 + r) * 128, 8)
            for h in range(NH):
                pltpu.async_copy(
                    stag.at[pl.ds(h * SWIN * 128 + so, 128)],
                    out_hbm.at[pl.ds(h * NVOX * 128 + wo + so, 128)], semf)
            return 0

        lax.fori_loop(0, cnt - n8 * 8, f1, 0)

        for k in range(nvec):
            rows[pl.ds(k * 16, 16)] = pre[k]
        pid_v[pl.ds(0, 16)] = jnp.full((16,), previd, jnp.int32)
        pltpu.async_copy(rows.at[pl.ds(0, D)],
                         pre_hbm.at[pl.ds(wid * D, D)], semf)
        pltpu.async_copy(pid_v.at[pl.ds(0, 16)],
                         preid_hbm.at[pl.ds(wid * 16, 16)], semf)

        def d8(g, _):
            for h in range(NH):
                pltpu.make_async_copy(stag.at[pl.ds(0, 1024)],
                                      out_hbm.at[pl.ds(0, 1024)], semf).wait()
            return 0

        lax.fori_loop(0, n8, d8, 0)

        def d1(r, _):
            for h in range(NH):
                pltpu.make_async_copy(stag.at[pl.ds(0, 128)],
                                      out_hbm.at[pl.ds(0, 128)], semf).wait()
            return 0

        lax.fori_loop(0, cnt - n8 * 8, d1, 0)
        pltpu.make_async_copy(rows.at[pl.ds(0, D)],
                              pre_hbm.at[pl.ds(0, D)], semf).wait()
        pltpu.make_async_copy(pid_v.at[pl.ds(0, 16)],
                              preid_hbm.at[pl.ds(0, 16)], semf).wait()

    return segmax


_segmax128 = _make_segmax(128)
_segmax256 = _make_segmax(256)


def _gather_kernel(idx2_hbm, table_hbm, out_hbm, idx_v, rows, g0, g1, p0, p1):
    wid = lax.axis_index("s") * 2 + lax.axis_index("c")
    base = pl.multiple_of(wid * CHUNK, 64)
    pltpu.sync_copy(idx2_hbm.at[wid], idx_v)
    gsems = (g0, g1)
    psems = (p0, p1)

    def _start(g, buf):
        pltpu.async_copy(table_hbm.at[idx_v.at[g]], rows.at[buf], gsems[buf])

    def _finish(g, buf):
        pltpu.make_async_copy(table_hbm.at[idx_v.at[0]], rows.at[buf],
                              gsems[buf]).wait()
        pltpu.async_copy(rows.at[buf], out_hbm.at[pl.ds(base + g * GB, GB)],
                         psems[buf])

    _start(0, 0)
    _start(1, 1)

    def body(h, _):
        for par in range(2):
            g = 2 * h + par
            _finish(g, par)
            pltpu.make_async_copy(rows.at[par], out_hbm.at[pl.ds(0, GB)],
                                  psems[par]).wait()

            @pl.when(g + 2 < GCH)
            def _():
                _start(g + 2, par)
        return 0

    lax.fori_loop(0, GCH // 2, body, 0)


def _gather(idx2, table):
    mesh = plsc.VectorSubcoreMesh(core_axis_name="c", subcore_axis_name="s")
    k = functools.partial(
        pl.kernel,
        out_type=jax.ShapeDtypeStruct((NPAD, 128), jnp.float32),
        mesh=mesh,
        compiler_params=pltpu.CompilerParams(use_tc_tiling_on_sc=False),
        scratch_types=[
            pltpu.VMEM((GCH, GB), jnp.int32),
            pltpu.VMEM((2, GB, 128), jnp.float32),
            pltpu.SemaphoreType.DMA,
            pltpu.SemaphoreType.DMA,
            pltpu.SemaphoreType.DMA,
            pltpu.SemaphoreType.DMA,
        ],
    )(_gather_kernel)
    return k(idx2, table)


def kernel(inp_feat, vox2point_idx, W1x, b1x, W2x, b2x, W1r, b1r, W2r, b2r,
           Wax, bax, War, bar, Wv1, bv1, W3, b3, W4, b4, Wv2, bv2):
    f32 = jnp.float32
    bf16 = jnp.bfloat16
    idx = vox2point_idx.astype(jnp.int32)
    idx_pad = jnp.concatenate([
        jnp.full((16,), -1, jnp.int32), idx,
        jnp.full((NPAD - NPTS,), NVOX, jnp.int32)])
    idx2 = jnp.minimum(idx_pad[16:], NVOX - 1).reshape(NW, GCH, GB)

    feat = jnp.zeros((NPAD, 6), f32).at[:NPTS].set(inp_feat)
    z34 = jnp.zeros((3, 64), f32)
    w1x6 = jnp.concatenate([W1x.astype(f32), z34], axis=0)
    w1r6 = jnp.concatenate([z34, W1r.astype(f32)], axis=0)

    r2 = lambda b: b.reshape(1, -1).astype(f32)

    pf2 = _stage1(feat, w1x6, r2(b1x), W2x, r2(b2x), w1r6, r2(b1r),
                  W2r, r2(b2r), Wax, r2(bax), War, r2(bar))

    segraw, pre1, preid1 = _segmax128(idx_pad, pf2.reshape(-1))
    segraw = segraw.reshape(NVOX, 128)
    ids1 = preid1.reshape(NW, 16)
    seg = _dense_corr(segraw, pre1.reshape(NW, 128), ids1[:, :1],
                      ids1[:, 0].reshape(1, NW), Wv1, r2(bv1), 1000)

    pg = _gather(idx2, seg)

    pf5 = _stage3(pg, pf2, W3[:128].astype(bf16), W3[128:].astype(bf16),
                  r2(b3), W4.astype(bf16), r2(b4))

    seg2f, pre2, preid2 = _segmax256(idx_pad, pf5.reshape(-1))
    seg2 = seg2f.reshape(2, NVOX, 128)
    pre2m = pre2.reshape(NW, 256)
    ids2 = preid2.reshape(NW, 16)
    out = _dense_corr2(seg2[0], seg2[1], pre2m[:, :128], pre2m[:, 128:],
                       ids2[:, :1], ids2[:, 0].reshape(1, NW),
                       Wv2[:128], Wv2[128:], r2(bv2), 1000)
    return out
